# baseline (speedup 1.0000x reference)
import functools

import jax
import jax.numpy as jnp
from jax import lax
from jax.experimental import pallas as pl
from jax.experimental.pallas import tpu as pltpu
from jax.experimental.pallas import tpu_sc as plsc

N = 100000
D = 128
G = 128
H = N // 2
B = 10000
NBH = H // B
NW = 32
C = 1568
H_PAD = C * NW
ACC = 256


def _tc_body(x_ref, w1_ref, b1_ref, w2_ref, b2_ref, e_ref):
    h = jnp.dot(x_ref[...], w1_ref[...], preferred_element_type=jnp.float32)
    h = h + b1_ref[...]
    h = h * (0.5 * jnp.tanh(0.5 * h) + 0.5)
    e = jnp.dot(h, w2_ref[...], preferred_element_type=jnp.float32)
    e_ref[...] = e + b2_ref[...]


def _tc_mlp(xh, W1, b1r, W2, b2r):
    return pl.pallas_call(
        _tc_body,
        grid=(NBH,),
        in_specs=[
            pl.BlockSpec((B, D), lambda i: (i, 0)),
            pl.BlockSpec((D, D), lambda i: (0, 0)),
            pl.BlockSpec((1, D), lambda i: (0, 0)),
            pl.BlockSpec((D, 1), lambda i: (0, 0)),
            pl.BlockSpec((1, 1), lambda i: (0, 0)),
        ],
        out_specs=pl.BlockSpec((B, 1), lambda i: (i, 0)),
        out_shape=jax.ShapeDtypeStruct((H_PAD, 1), jnp.float32),
    )(xh, W1, b1r, W2, b2r)


_sc_mesh = plsc.VectorSubcoreMesh(core_axis_name="c", subcore_axis_name="s")


@functools.partial(
    pl.kernel,
    out_type=jax.ShapeDtypeStruct((2, G), jnp.float32),
    mesh=_sc_mesh,
    scratch_types=[
        pltpu.VMEM((C,), jnp.float32),
        pltpu.VMEM((C,), jnp.int32),
        pltpu.VMEM((ACC,), jnp.float32),
        pltpu.VMEM((G,), jnp.int32),
        pltpu.VMEM((G,), jnp.float32),
        pltpu.VMEM_SHARED((G,), jnp.float32),
    ],
    compiler_params=pltpu.CompilerParams(needs_layout_passes=False),
)
def _sc_segsum(e_hbm, seg_hbm, out_hbm, e_v, seg_v, acc_v, idx_v, zero_v,
               shared_acc):
    cid = lax.axis_index("c")
    sid = lax.axis_index("s")
    wid = sid * 2 + cid
    base = wid * C

    pltpu.sync_copy(e_hbm.at[pl.ds(base, C)], e_v)
    pltpu.sync_copy(seg_hbm.at[pl.ds(base, C)], seg_v)

    lane = lax.iota(jnp.int32, 16)
    zeros16 = jnp.zeros((16,), jnp.float32)
    for i in range(ACC // 16):
        acc_v[pl.ds(i * 16, 16)] = zeros16
    for i in range(G // 16):
        idx_v[pl.ds(i * 16, 16)] = lane + 16 * i
        zero_v[pl.ds(i * 16, 16)] = zeros16

    def body(i, carry):
        s = i * 16
        ids = seg_v[pl.ds(s, 16)]
        vals = e_v[pl.ds(s, 16)]
        plsc.addupdate_scatter(acc_v, [ids], vals)
        return carry

    lax.fori_loop(0, C // 16, body, 0)

    @pl.when(sid == 0)
    def _():
        pltpu.sync_copy(zero_v, shared_acc)

    plsc.subcore_barrier()
    pltpu.sync_copy(acc_v.at[pl.ds(0, G)], shared_acc.at[idx_v], add=True)
    plsc.subcore_barrier()

    @pl.when(sid == 0)
    def _():
        pltpu.sync_copy(shared_acc, out_hbm.at[cid])


def kernel(x, atomic_numbers, batch_segments, graph_mask, W1, b1, W2, b2):
    x2 = x.reshape(N, D)
    b1r = b1.reshape(1, D)
    b2r = b2.reshape(1, 1)
    seg = batch_segments.astype(jnp.int32)
    pad = jnp.full((H_PAD - H,), G, dtype=jnp.int32)
    seg1 = jnp.concatenate([seg[:H], pad])
    seg2 = jnp.concatenate([seg[H:], pad])

    e1 = _tc_mlp(x2[:H], W1, b1r, W2, b2r).reshape(H_PAD)
    p1 = _sc_segsum(e1, seg1)
    e2 = _tc_mlp(x2[H:], W1, b1r, W2, b2r).reshape(H_PAD)
    p2 = _sc_segsum(e2, seg2)

    energy = jnp.where(graph_mask, p1[0] + p1[1] + p2[0] + p2[1], 0.0)
    return (-jnp.sum(energy), energy)


# speedup vs baseline: 1.4805x; 1.4805x over previous
import functools

import jax
import jax.numpy as jnp
from jax import lax
from jax.experimental import pallas as pl
from jax.experimental.pallas import tpu as pltpu
from jax.experimental.pallas import tpu_sc as plsc

N = 100000
D = 128
G = 128
B = 25000
NB = N // B
NW = 32
C = 3136
N_PAD = C * NW
ACC = 256


def _tc_body(x_ref, w1_ref, b1_ref, w2_ref, b2_ref, e_ref):
    h = jnp.dot(x_ref[...], w1_ref[...], preferred_element_type=jnp.float32)
    h = h + b1_ref[...]
    h = h * (0.5 * jnp.tanh(0.5 * h) + 0.5)
    e = jnp.dot(h, w2_ref[...], preferred_element_type=jnp.float32)
    e_ref[...] = e + b2_ref[...]


@jax.jit
def _tc_mlp(x2, W1, b1r, W2, b2r):
    return pl.pallas_call(
        _tc_body,
        grid=(NB,),
        in_specs=[
            pl.BlockSpec((B, D), lambda i: (i, 0)),
            pl.BlockSpec((D, D), lambda i: (0, 0)),
            pl.BlockSpec((1, D), lambda i: (0, 0)),
            pl.BlockSpec((D, 1), lambda i: (0, 0)),
            pl.BlockSpec((1, 1), lambda i: (0, 0)),
        ],
        out_specs=pl.BlockSpec((B, 1), lambda i: (i, 0)),
        out_shape=jax.ShapeDtypeStruct((N_PAD, 1), jnp.float32),
    )(x2, W1, b1r, W2, b2r)


_sc_mesh = plsc.VectorSubcoreMesh(core_axis_name="c", subcore_axis_name="s")


@functools.partial(
    pl.kernel,
    out_type=jax.ShapeDtypeStruct((2, G), jnp.float32),
    mesh=_sc_mesh,
    scratch_types=[
        pltpu.VMEM((C,), jnp.float32),
        pltpu.VMEM((C,), jnp.int32),
        pltpu.VMEM((ACC,), jnp.float32),
        pltpu.VMEM((G,), jnp.int32),
        pltpu.VMEM((G,), jnp.float32),
        pltpu.VMEM_SHARED((G,), jnp.float32),
    ],
    compiler_params=pltpu.CompilerParams(needs_layout_passes=False),
)
def _sc_segsum(e_hbm, seg_hbm, out_hbm, e_v, seg_v, acc_v, idx_v, zero_v,
               shared_acc):
    cid = lax.axis_index("c")
    sid = lax.axis_index("s")
    wid = sid * 2 + cid
    base = wid * C

    pltpu.sync_copy(e_hbm.at[pl.ds(base, C)], e_v)
    pltpu.sync_copy(seg_hbm.at[pl.ds(base, C)], seg_v)

    lane = lax.iota(jnp.int32, 16)
    zeros16 = jnp.zeros((16,), jnp.float32)
    for i in range(ACC // 16):
        acc_v[pl.ds(i * 16, 16)] = zeros16
    for i in range(G // 16):
        idx_v[pl.ds(i * 16, 16)] = lane + 16 * i
        zero_v[pl.ds(i * 16, 16)] = zeros16

    def body(i, carry):
        s = i * 16
        ids = seg_v[pl.ds(s, 16)]
        vals = e_v[pl.ds(s, 16)]
        plsc.addupdate_scatter(acc_v, [ids], vals)
        return carry

    lax.fori_loop(0, C // 16, body, 0)

    @pl.when(sid == 0)
    def _():
        pltpu.sync_copy(zero_v, shared_acc)

    plsc.subcore_barrier()
    pltpu.sync_copy(acc_v.at[pl.ds(0, G)], shared_acc.at[idx_v], add=True)
    plsc.subcore_barrier()

    @pl.when(sid == 0)
    def _():
        pltpu.sync_copy(shared_acc, out_hbm.at[cid])


def kernel(x, atomic_numbers, batch_segments, graph_mask, W1, b1, W2, b2):
    x2 = x.reshape(N, D)
    b1r = b1.reshape(1, D)
    b2r = b2.reshape(1, 1)
    e = _tc_mlp(x2, W1, b1r, W2, b2r).reshape(N_PAD)
    seg_pad = jnp.concatenate(
        [batch_segments.astype(jnp.int32),
         jnp.full((N_PAD - N,), G, dtype=jnp.int32)])
    partials = _sc_segsum(e, seg_pad)
    energy = jnp.where(graph_mask, partials[0] + partials[1], 0.0)
    return (-jnp.sum(energy), energy)


# speedup vs baseline: 2.4084x; 1.6267x over previous
import functools

import jax
import jax.numpy as jnp
from jax import lax
from jax.experimental import pallas as pl
from jax.experimental.pallas import tpu as pltpu
from jax.experimental.pallas import tpu_sc as plsc

N = 100000
D = 128
G = 128
B = 20480
N_PAD = 102400
NB = N_PAD // B
NW = 32
C = N_PAD // NW
ACC = 256


def _tc_body(x_ref, w1_ref, b1_ref, w2_ref, b2_ref, e_ref):
    h = jnp.dot(x_ref[...], w1_ref[...], preferred_element_type=jnp.float32)
    h = h + b1_ref[...]
    h = h * (0.5 * jnp.tanh(0.5 * h) + 0.5)
    e = jnp.dot(h, w2_ref[...], preferred_element_type=jnp.float32)
    e = e + b2_ref[...]
    e_ref[...] = e.reshape(B // 128, 128)


@jax.jit
def _tc_mlp(x2, W1, b1r, W2, b2r):
    return pl.pallas_call(
        _tc_body,
        grid=(NB,),
        in_specs=[
            pl.BlockSpec((B, D), lambda i: (i, 0)),
            pl.BlockSpec((D, D), lambda i: (0, 0)),
            pl.BlockSpec((1, D), lambda i: (0, 0)),
            pl.BlockSpec((D, 1), lambda i: (0, 0)),
            pl.BlockSpec((1, 1), lambda i: (0, 0)),
        ],
        out_specs=pl.BlockSpec((B // 128, 128), lambda i: (i, 0)),
        out_shape=jax.ShapeDtypeStruct((N_PAD // 128, 128), jnp.float32),
    )(x2, W1, b1r, W2, b2r)


_sc_mesh = plsc.VectorSubcoreMesh(core_axis_name="c", subcore_axis_name="s")


@functools.partial(
    pl.kernel,
    out_type=jax.ShapeDtypeStruct((2, G), jnp.float32),
    mesh=_sc_mesh,
    scratch_types=[
        pltpu.VMEM((C,), jnp.float32),
        pltpu.VMEM((C,), jnp.int32),
        pltpu.VMEM((ACC,), jnp.float32),
        pltpu.VMEM((G,), jnp.int32),
        pltpu.VMEM((G,), jnp.float32),
        pltpu.VMEM_SHARED((G,), jnp.float32),
    ],
    compiler_params=pltpu.CompilerParams(needs_layout_passes=False),
)
def _sc_segsum(e_hbm, seg_hbm, out_hbm, e_v, seg_v, acc_v, idx_v, zero_v,
               shared_acc):
    cid = lax.axis_index("c")
    sid = lax.axis_index("s")
    wid = sid * 2 + cid
    base = wid * C

    pltpu.sync_copy(e_hbm.at[pl.ds(base, C)], e_v)
    pltpu.sync_copy(seg_hbm.at[pl.ds(base, C)], seg_v)

    lane = lax.iota(jnp.int32, 16)
    zeros16 = jnp.zeros((16,), jnp.float32)
    for i in range(ACC // 16):
        acc_v[pl.ds(i * 16, 16)] = zeros16
    for i in range(G // 16):
        idx_v[pl.ds(i * 16, 16)] = lane + 16 * i
        zero_v[pl.ds(i * 16, 16)] = zeros16

    def body(i, carry):
        s = i * 16
        ids = seg_v[pl.ds(s, 16)]
        vals = e_v[pl.ds(s, 16)]
        plsc.addupdate_scatter(acc_v, [ids], vals)
        return carry

    lax.fori_loop(0, C // 16, body, 0)

    @pl.when(sid == 0)
    def _():
        pltpu.sync_copy(zero_v, shared_acc)

    plsc.subcore_barrier()
    pltpu.sync_copy(acc_v.at[pl.ds(0, G)], shared_acc.at[idx_v], add=True)
    plsc.subcore_barrier()

    @pl.when(sid == 0)
    def _():
        pltpu.sync_copy(shared_acc, out_hbm.at[cid])


def kernel(x, atomic_numbers, batch_segments, graph_mask, W1, b1, W2, b2):
    x2 = x.reshape(N, D)
    b1r = b1.reshape(1, D)
    b2r = b2.reshape(1, 1)
    e = _tc_mlp(x2, W1, b1r, W2, b2r).reshape(N_PAD)
    seg_pad = jnp.concatenate(
        [batch_segments.astype(jnp.int32),
         jnp.full((N_PAD - N,), G, dtype=jnp.int32)])
    partials = _sc_segsum(e, seg_pad)
    energy = jnp.where(graph_mask, partials[0] + partials[1], 0.0)
    return (-jnp.sum(energy), energy)
